# baseline (speedup 1.0000x reference)
import functools

import jax
import jax.numpy as jnp
from jax import lax
from jax.experimental import pallas as pl
from jax.experimental.pallas import tpu as pltpu
from jax.experimental.pallas import tpu_sc as plsc

S = 10000
SH = 5000
D = 128
N = 320000
BLK = 128
NBLK = N // BLK
NC = 2
NS = 16
NPAIR = 79
HROWS = 2
NHEAD = NS * HROWS * BLK
U_SUB = 312
U_TAIL = SH - NS * U_SUB

_mesh = plsc.VectorSubcoreMesh(core_axis_name="c", subcore_axis_name="s")


@functools.partial(
    pl.kernel,
    out_type=jax.ShapeDtypeStruct((S, D), jnp.float32),
    mesh=_mesh,
    scratch_types=[
        pltpu.VMEM_SHARED((SH + 8, D), jnp.float32),
        pltpu.VMEM_SHARED((NS, 8, 16), jnp.int32),
        pltpu.VMEM((2, BLK, D), jnp.float32),
        pltpu.VMEM((2, 1, BLK), jnp.int32),
        pltpu.VMEM((2, 1, BLK), jnp.int32),
        pltpu.VMEM((HROWS, BLK), jnp.int32),
        pltpu.VMEM((8, 16), jnp.int32),
        pltpu.VMEM((NS, 8, 16), jnp.int32),
        pltpu.SemaphoreType.DMA((2,)),
        pltpu.SemaphoreType.DMA,
    ],
)
def _segsum(u_hbm, v_hbm, idx_hbm, heads_hbm, out_hbm, acc, cnts, vbuf, ibuf,
            sbuf, cibuf, cbuf, crbuf, gsem, usem):
    c = lax.axis_index("c")
    s = lax.axis_index("s")
    ubase = s * U_SUB
    urow = c * SH + ubase

    pltpu.async_copy(u_hbm.at[pl.ds(urow, U_SUB)],
                     acc.at[pl.ds(ubase, U_SUB)], usem)

    @pl.when(s == NS - 1)
    def _():
        pltpu.async_copy(u_hbm.at[pl.ds(c * SH + NS * U_SUB, U_TAIL)],
                         acc.at[pl.ds(NS * U_SUB, U_TAIL)], usem)

    pltpu.sync_copy(heads_hbm.at[s], cibuf)
    lim = jnp.full((16,), SH, jnp.int32)
    one = jnp.full((16,), 1, jnp.int32)
    zero = jnp.zeros((16,), jnp.int32)

    tot = zero
    for r in range(HROWS):
        for k in range(BLK // 16):
            seg = cibuf[r, pl.ds(16 * k, 16)]
            tot = tot + jnp.where(seg < lim, one, zero)
    cbuf[0, :] = tot
    pltpu.sync_copy(cbuf, cnts.at[s])

    pltpu.make_async_copy(u_hbm.at[pl.ds(urow, U_SUB)],
                          acc.at[pl.ds(ubase, U_SUB)], usem).wait()

    @pl.when(s == NS - 1)
    def _():
        pltpu.make_async_copy(u_hbm.at[pl.ds(c * SH + NS * U_SUB, U_TAIL)],
                              acc.at[pl.ds(NS * U_SUB, U_TAIL)], usem).wait()

    plsc.subcore_barrier()

    pltpu.sync_copy(cnts, crbuf)
    tvec = zero
    for t2 in range(NS):
        tvec = tvec + crbuf[t2, 0, pl.ds(0, 16)]
    big_h = tvec[0]
    for j in range(1, 16):
        big_h = big_h + tvec[j]

    first1 = jnp.maximum(big_h - 1, 0)
    first = jnp.where(c == 0, 0, first1)
    nblk = jnp.where(c == 0, big_h, NBLK - first1)
    segbase = jnp.full((16,), c * SH, jnp.int32)
    guard = jnp.full((16,), SH, jnp.int32)

    def fire(t, b):
        j = s + t * NS

        @pl.when(j < nblk)
        def _():
            blk = first + j
            pltpu.async_copy(idx_hbm.at[blk], ibuf.at[b], gsem.at[b])
            pltpu.async_copy(v_hbm.at[pl.ds(blk * BLK, BLK)],
                             vbuf.at[b], gsem.at[b])

    def consume(t, b):
        j = s + t * NS

        @pl.when(j < nblk)
        def _():
            blk = first + j
            pltpu.make_async_copy(v_hbm.at[pl.ds(blk * BLK, BLK)],
                                  vbuf.at[b], gsem.at[b]).wait()
            pltpu.make_async_copy(idx_hbm.at[blk], ibuf.at[b],
                                  gsem.at[b]).wait()
            for k in range(BLK // 16):
                seg = ibuf[b, 0, pl.ds(16 * k, 16)]
                loc = seg - segbase
                ok = jnp.logical_and(loc >= zero, loc < lim)
                sbuf[b, 0, pl.ds(16 * k, 16)] = jnp.where(ok, loc, guard)
            pltpu.sync_copy(vbuf.at[b], acc.at[sbuf.at[b, 0]], add=True)

    fire(0, 0)

    def mbody(i, carry):
        t0 = 2 * i
        fire(t0 + 1, 1)
        consume(t0, 0)
        fire(t0 + 2, 0)
        consume(t0 + 1, 1)
        return carry

    lax.fori_loop(0, NPAIR, mbody, 0)
    plsc.subcore_barrier()

    pltpu.sync_copy(acc.at[pl.ds(ubase, U_SUB)],
                    out_hbm.at[pl.ds(urow, U_SUB)])

    @pl.when(s == NS - 1)
    def _():
        pltpu.sync_copy(acc.at[pl.ds(NS * U_SUB, U_TAIL)],
                        out_hbm.at[pl.ds(c * SH + NS * U_SUB, U_TAIL)])


def kernel(u, v, batch):
    idx = batch.astype(jnp.int32).reshape(NBLK, 1, BLK)
    heads = jnp.pad(idx[:, 0, 0], (0, NHEAD - NBLK),
                    constant_values=S).reshape(NS, HROWS, BLK)
    return _segsum(u, v, idx, heads)


# speedup vs baseline: 1.3232x; 1.3232x over previous
import functools

import jax
import jax.numpy as jnp
from jax import lax
from jax.experimental import pallas as pl
from jax.experimental.pallas import tpu as pltpu
from jax.experimental.pallas import tpu_sc as plsc

S = 10000
SH = 5000
D = 128
N = 320000
BLK = 128
NBLK = N // BLK
NC = 2
NS = 16
NITW = 157
NPAIR = 79
CROWS = 157
IDX_PAD = NS * CROWS
U_SUB = 312
U_TAIL = SH - NS * U_SUB

_mesh = plsc.VectorSubcoreMesh(core_axis_name="c", subcore_axis_name="s")


@functools.partial(
    pl.kernel,
    out_type=jax.ShapeDtypeStruct((S, D), jnp.float32),
    mesh=_mesh,
    scratch_types=[
        pltpu.VMEM_SHARED((SH + 8, D), jnp.float32),
        pltpu.VMEM_SHARED((NS, 8, 16), jnp.int32),
        pltpu.VMEM((2, BLK, D), jnp.float32),
        pltpu.VMEM((2, 1, BLK), jnp.int32),
        pltpu.VMEM((2, 1, BLK), jnp.int32),
        pltpu.VMEM((CROWS, 1, BLK), jnp.int32),
        pltpu.VMEM((8, 16), jnp.int32),
        pltpu.VMEM((NS, 8, 16), jnp.int32),
        pltpu.SemaphoreType.DMA((2,)),
        pltpu.SemaphoreType.DMA,
    ],
)
def _segsum(u_hbm, v_hbm, idx_hbm, out_hbm, acc, cnts, vbuf, ibuf, sbuf,
            cibuf, cbuf, crbuf, gsem, usem):
    c = lax.axis_index("c")
    s = lax.axis_index("s")
    ubase = s * U_SUB
    urow = c * SH + ubase

    pltpu.async_copy(u_hbm.at[pl.ds(urow, U_SUB)],
                     acc.at[pl.ds(ubase, U_SUB)], usem)

    @pl.when(s == NS - 1)
    def _():
        pltpu.async_copy(u_hbm.at[pl.ds(c * SH + NS * U_SUB, U_TAIL)],
                         acc.at[pl.ds(NS * U_SUB, U_TAIL)], usem)

    pltpu.sync_copy(idx_hbm.at[pl.ds(s * CROWS, CROWS)], cibuf)
    lim = jnp.full((16,), SH, jnp.int32)
    one = jnp.full((16,), 1, jnp.int32)
    zero = jnp.zeros((16,), jnp.int32)

    tot = zero
    for r in range(CROWS):
        for k in range(BLK // 16):
            seg = cibuf[r, 0, pl.ds(16 * k, 16)]
            tot = tot + jnp.where(seg < lim, one, zero)
    cbuf[0, :] = tot
    for k in range(NS):
        @pl.when(s == k)
        def _():
            pltpu.sync_copy(cbuf, cnts.at[k])

    pltpu.make_async_copy(u_hbm.at[pl.ds(urow, U_SUB)],
                          acc.at[pl.ds(ubase, U_SUB)], usem).wait()

    @pl.when(s == NS - 1)
    def _():
        pltpu.make_async_copy(u_hbm.at[pl.ds(c * SH + NS * U_SUB, U_TAIL)],
                              acc.at[pl.ds(NS * U_SUB, U_TAIL)], usem).wait()

    plsc.subcore_barrier()

    pltpu.sync_copy(cnts, crbuf)
    tvec = zero
    for t2 in range(NS):
        tvec = tvec + crbuf[t2, 0, pl.ds(0, 16)]
    big_t = tvec[0]
    for j in range(1, 16):
        big_t = big_t + tvec[j]

    first = jnp.where(c == 0, 0, big_t // BLK)
    nblk = jnp.where(c == 0, (big_t + BLK - 1) // BLK, NBLK - big_t // BLK)
    segbase = jnp.full((16,), c * SH, jnp.int32)
    guard = jnp.full((16,), SH, jnp.int32)

    def fire(t, b):
        j = s + t * NS

        @pl.when(j < nblk)
        def _():
            blk = first + j
            pltpu.async_copy(idx_hbm.at[blk], ibuf.at[b], gsem.at[b])
            pltpu.async_copy(v_hbm.at[pl.ds(blk * BLK, BLK)],
                             vbuf.at[b], gsem.at[b])

    def consume(t, b):
        j = s + t * NS

        @pl.when(j < nblk)
        def _():
            blk = first + j
            pltpu.make_async_copy(v_hbm.at[pl.ds(blk * BLK, BLK)],
                                  vbuf.at[b], gsem.at[b]).wait()
            pltpu.make_async_copy(idx_hbm.at[blk], ibuf.at[b],
                                  gsem.at[b]).wait()
            for k in range(BLK // 16):
                seg = ibuf[b, 0, pl.ds(16 * k, 16)]
                loc = seg - segbase
                ok = jnp.logical_and(loc >= zero, loc < lim)
                sbuf[b, 0, pl.ds(16 * k, 16)] = jnp.where(ok, loc, guard)
            pltpu.sync_copy(vbuf.at[b], acc.at[sbuf.at[b, 0]], add=True)

    fire(0, 0)

    def mbody(i, carry):
        t0 = 2 * i
        fire(t0 + 1, 1)
        consume(t0, 0)
        fire(t0 + 2, 0)
        consume(t0 + 1, 1)
        return carry

    lax.fori_loop(0, NPAIR, mbody, 0)
    plsc.subcore_barrier()

    pltpu.sync_copy(acc.at[pl.ds(ubase, U_SUB)],
                    out_hbm.at[pl.ds(urow, U_SUB)])

    @pl.when(s == NS - 1)
    def _():
        pltpu.sync_copy(acc.at[pl.ds(NS * U_SUB, U_TAIL)],
                        out_hbm.at[pl.ds(c * SH + NS * U_SUB, U_TAIL)])


def kernel(u, v, batch):
    idx = batch.astype(jnp.int32).reshape(NBLK, BLK)
    idx = jnp.pad(idx, ((0, IDX_PAD - NBLK), (0, 0)),
                  constant_values=S).reshape(IDX_PAD, 1, BLK)
    return _segsum(u, v, idx)


# speedup vs baseline: 1.3277x; 1.0033x over previous
import functools

import jax
import jax.numpy as jnp
from jax import lax
from jax.experimental import pallas as pl
from jax.experimental.pallas import tpu as pltpu
from jax.experimental.pallas import tpu_sc as plsc

S = 10000
SH = 5000
D = 128
N = 320000
BLK = 128
NBLK = N // BLK
NC = 2
NS = 16
NPAIR = 79
CROWS = 157
IDX_PAD = NS * CROWS
U_SUB = 312
U_TAIL = SH - NS * U_SUB

_mesh = plsc.VectorSubcoreMesh(core_axis_name="c", subcore_axis_name="s")


@functools.partial(
    pl.kernel,
    out_type=jax.ShapeDtypeStruct((S, D), jnp.float32),
    mesh=_mesh,
    scratch_types=[
        pltpu.VMEM_SHARED((SH + 8, D), jnp.float32),
        pltpu.VMEM_SHARED((NS, 8, 16), jnp.int32),
        pltpu.VMEM((2, BLK, D), jnp.float32),
        pltpu.VMEM((2, 1, BLK), jnp.int32),
        pltpu.VMEM((2, 1, BLK), jnp.int32),
        pltpu.VMEM((CROWS, 1, BLK), jnp.int32),
        pltpu.VMEM((8, 16), jnp.int32),
        pltpu.VMEM((NS, 8, 16), jnp.int32),
        pltpu.SemaphoreType.DMA((2,)),
        pltpu.SemaphoreType.DMA,
    ],
)
def _segsum(u_hbm, v_hbm, idx_hbm, out_hbm, acc, cnts, vbuf, ibuf, sbuf,
            cibuf, cbuf, crbuf, gsem, usem):
    c = lax.axis_index("c")
    s = lax.axis_index("s")
    ubase = s * U_SUB
    urow = c * SH + ubase

    pltpu.async_copy(u_hbm.at[pl.ds(urow, U_SUB)],
                     acc.at[pl.ds(ubase, U_SUB)], usem)

    @pl.when(s == NS - 1)
    def _():
        pltpu.async_copy(u_hbm.at[pl.ds(c * SH + NS * U_SUB, U_TAIL)],
                         acc.at[pl.ds(NS * U_SUB, U_TAIL)], usem)

    pltpu.sync_copy(idx_hbm.at[pl.ds(s * CROWS, CROWS)], cibuf)
    lim = jnp.full((16,), SH, jnp.int32)
    one = jnp.full((16,), 1, jnp.int32)
    zero = jnp.zeros((16,), jnp.int32)

    tot = zero
    for r in range(CROWS):
        for k in range(BLK // 16):
            seg = cibuf[r, 0, pl.ds(16 * k, 16)]
            tot = tot + jnp.where(seg < lim, one, zero)
    cbuf[0, :] = tot
    for k in range(NS):
        @pl.when(s == k)
        def _():
            pltpu.sync_copy(cbuf, cnts.at[k])

    pltpu.make_async_copy(u_hbm.at[pl.ds(urow, U_SUB)],
                          acc.at[pl.ds(ubase, U_SUB)], usem).wait()

    @pl.when(s == NS - 1)
    def _():
        pltpu.make_async_copy(u_hbm.at[pl.ds(c * SH + NS * U_SUB, U_TAIL)],
                              acc.at[pl.ds(NS * U_SUB, U_TAIL)], usem).wait()

    plsc.subcore_barrier()

    pltpu.sync_copy(cnts, crbuf)
    tvec = zero
    for t2 in range(NS):
        tvec = tvec + crbuf[t2, 0, pl.ds(0, 16)]
    big_t = tvec[0]
    for j in range(1, 16):
        big_t = big_t + tvec[j]

    first = jnp.where(c == 0, 0, big_t // BLK)
    nblk = jnp.where(c == 0, (big_t + BLK - 1) // BLK, NBLK - big_t // BLK)
    segbase = jnp.full((16,), c * SH, jnp.int32)
    guard = jnp.full((16,), SH, jnp.int32)

    def fire(t, b):
        j = s + t * NS

        @pl.when(j < nblk)
        def _():
            blk = first + j
            pltpu.async_copy(idx_hbm.at[blk], ibuf.at[b], gsem.at[b])
            pltpu.async_copy(v_hbm.at[pl.ds(blk * BLK, BLK)],
                             vbuf.at[b], gsem.at[b])

    def consume(t, b):
        j = s + t * NS

        @pl.when(j < nblk)
        def _():
            blk = first + j
            pltpu.make_async_copy(v_hbm.at[pl.ds(blk * BLK, BLK)],
                                  vbuf.at[b], gsem.at[b]).wait()
            pltpu.make_async_copy(idx_hbm.at[blk], ibuf.at[b],
                                  gsem.at[b]).wait()
            for k in range(BLK // 16):
                seg = ibuf[b, 0, pl.ds(16 * k, 16)]
                loc = seg - segbase
                ok = jnp.logical_and(loc >= zero, loc < lim)
                sbuf[b, 0, pl.ds(16 * k, 16)] = jnp.where(ok, loc, guard)
            pltpu.sync_copy(vbuf.at[b], acc.at[sbuf.at[b, 0]], add=True)

    fire(0, 0)

    def mbody(i, carry):
        t0 = 2 * i
        fire(t0 + 1, 1)
        consume(t0, 0)
        fire(t0 + 2, 0)
        consume(t0 + 1, 1)
        return carry

    lax.fori_loop(0, NPAIR, mbody, 0)
    plsc.subcore_barrier()

    pltpu.sync_copy(acc.at[pl.ds(ubase, U_SUB)],
                    out_hbm.at[pl.ds(urow, U_SUB)])

    @pl.when(s == NS - 1)
    def _():
        pltpu.sync_copy(acc.at[pl.ds(NS * U_SUB, U_TAIL)],
                        out_hbm.at[pl.ds(c * SH + NS * U_SUB, U_TAIL)])


def kernel(u, v, batch):
    idx = batch.astype(jnp.int32).reshape(NBLK, BLK)
    idx = jnp.pad(idx, ((0, IDX_PAD - NBLK), (0, 0)),
                  constant_values=S).reshape(IDX_PAD, 1, BLK)
    return _segsum(u, v, idx)


# speedup vs baseline: 1.3480x; 1.0153x over previous
import functools

import jax
import jax.numpy as jnp
from jax import lax
from jax.experimental import pallas as pl
from jax.experimental.pallas import tpu as pltpu
from jax.experimental.pallas import tpu_sc as plsc

S = 10000
SH = 5000
D = 128
N = 320000
BLK = 128
NBLK = N // BLK
NC = 2
NS = 16
NPAIR = 79
CROWS = 157
IDX_PAD = NS * CROWS
U_SUB = 312
U_TAIL = SH - NS * U_SUB

_mesh = plsc.VectorSubcoreMesh(core_axis_name="c", subcore_axis_name="s")


@functools.partial(
    pl.kernel,
    out_type=jax.ShapeDtypeStruct((S, D), jnp.float32),
    mesh=_mesh,
    scratch_types=[
        pltpu.VMEM_SHARED((SH + 8, D), jnp.float32),
        pltpu.VMEM_SHARED((NS, 8, 16), jnp.int32),
        pltpu.VMEM((2, BLK, D), jnp.float32),
        pltpu.VMEM((2, 1, BLK), jnp.int32),
        pltpu.VMEM((2, 1, BLK), jnp.int32),
        pltpu.VMEM((CROWS, 1, BLK), jnp.int32),
        pltpu.VMEM((8, 16), jnp.int32),
        pltpu.VMEM((NS, 8, 16), jnp.int32),
        pltpu.SemaphoreType.DMA((2,)),
        pltpu.SemaphoreType.DMA,
    ],
)
def _segsum(u_hbm, v_hbm, idx_hbm, out_hbm, acc, cnts, vbuf, ibuf, sbuf,
            cibuf, cbuf, crbuf, gsem, usem):
    c = lax.axis_index("c")
    s = lax.axis_index("s")
    ubase = s * U_SUB
    urow = c * SH + ubase

    pltpu.async_copy(u_hbm.at[pl.ds(urow, U_SUB)],
                     acc.at[pl.ds(ubase, U_SUB)], usem)

    @pl.when(s == NS - 1)
    def _():
        pltpu.async_copy(u_hbm.at[pl.ds(c * SH + NS * U_SUB, U_TAIL)],
                         acc.at[pl.ds(NS * U_SUB, U_TAIL)], usem)

    pltpu.sync_copy(idx_hbm.at[pl.ds(s * CROWS, CROWS)], cibuf)
    lim = jnp.full((16,), SH, jnp.int32)
    one = jnp.full((16,), 1, jnp.int32)
    zero = jnp.zeros((16,), jnp.int32)

    tot = zero
    for r in range(CROWS):
        seg = cibuf[r, 0, pl.ds(0, 16)]
        tot = tot + jnp.where(seg < lim, one, zero)
    cbuf[0, :] = tot
    for k in range(NS):
        @pl.when(s == k)
        def _():
            pltpu.sync_copy(cbuf, cnts.at[k])

    pltpu.make_async_copy(u_hbm.at[pl.ds(urow, U_SUB)],
                          acc.at[pl.ds(ubase, U_SUB)], usem).wait()

    @pl.when(s == NS - 1)
    def _():
        pltpu.make_async_copy(u_hbm.at[pl.ds(c * SH + NS * U_SUB, U_TAIL)],
                              acc.at[pl.ds(NS * U_SUB, U_TAIL)], usem).wait()

    plsc.subcore_barrier()

    pltpu.sync_copy(cnts, crbuf)
    tvec = zero
    for t2 in range(NS):
        tvec = tvec + crbuf[t2, 0, pl.ds(0, 16)]
    big_h = tvec[0]

    first1 = jnp.maximum(big_h - 1, 0)
    first = jnp.where(c == 0, 0, first1)
    nblk = jnp.where(c == 0, big_h, NBLK - first1)
    segbase = jnp.full((16,), c * SH, jnp.int32)
    guard = jnp.full((16,), SH, jnp.int32)

    def fire(t, b):
        j = s + t * NS

        @pl.when(j < nblk)
        def _():
            blk = first + j
            pltpu.async_copy(idx_hbm.at[blk], ibuf.at[b], gsem.at[b])
            pltpu.async_copy(v_hbm.at[pl.ds(blk * BLK, BLK)],
                             vbuf.at[b], gsem.at[b])

    def consume(t, b):
        j = s + t * NS

        @pl.when(j < nblk)
        def _():
            blk = first + j
            pltpu.make_async_copy(v_hbm.at[pl.ds(blk * BLK, BLK)],
                                  vbuf.at[b], gsem.at[b]).wait()
            pltpu.make_async_copy(idx_hbm.at[blk], ibuf.at[b],
                                  gsem.at[b]).wait()
            for k in range(BLK // 16):
                seg = ibuf[b, 0, pl.ds(16 * k, 16)]
                loc = seg - segbase
                ok = jnp.logical_and(loc >= zero, loc < lim)
                sbuf[b, 0, pl.ds(16 * k, 16)] = jnp.where(ok, loc, guard)
            pltpu.sync_copy(vbuf.at[b], acc.at[sbuf.at[b, 0]], add=True)

    fire(0, 0)

    def mbody(i, carry):
        t0 = 2 * i
        fire(t0 + 1, 1)
        consume(t0, 0)
        fire(t0 + 2, 0)
        consume(t0 + 1, 1)
        return carry

    lax.fori_loop(0, NPAIR, mbody, 0)
    plsc.subcore_barrier()

    pltpu.sync_copy(acc.at[pl.ds(ubase, U_SUB)],
                    out_hbm.at[pl.ds(urow, U_SUB)])

    @pl.when(s == NS - 1)
    def _():
        pltpu.sync_copy(acc.at[pl.ds(NS * U_SUB, U_TAIL)],
                        out_hbm.at[pl.ds(c * SH + NS * U_SUB, U_TAIL)])


def kernel(u, v, batch):
    idx = batch.astype(jnp.int32).reshape(NBLK, BLK)
    idx = jnp.pad(idx, ((0, IDX_PAD - NBLK), (0, 0)),
                  constant_values=S).reshape(IDX_PAD, 1, BLK)
    return _segsum(u, v, idx)
